# baseline (speedup 1.0000x reference)
import functools

import jax
import jax.numpy as jnp
from jax import lax
from jax.experimental import pallas as pl
from jax.experimental.pallas import tpu as pltpu
from jax.experimental.pallas import tpu_sc as plsc

NC = 2
NS = 16
NW = NC * NS
CHUNK = 128
DW = 16


def _mesh():
    return plsc.VectorSubcoreMesh(core_axis_name="c", subcore_axis_name="s")


@functools.lru_cache(maxsize=None)
def _prop_sc(n_acc, dh, cpt):
    stripe = n_acc // NS
    NSLOT = 3
    assert cpt % NSLOT == 0 and cpt > NSLOT

    @functools.partial(
        pl.kernel,
        out_type=jax.ShapeDtypeStruct((NC, n_acc, dh), jnp.float32),
        mesh=_mesh(),
        scratch_types=[
            pltpu.VMEM((cpt, CHUNK), jnp.int32),
            pltpu.VMEM((cpt, CHUNK), jnp.int32),
        ]
        + [pltpu.VMEM((CHUNK, dh), jnp.float32)] * NSLOT
        + [pltpu.VMEM_SHARED((n_acc, dh), jnp.float32)]
        + [pltpu.SemaphoreType.DMA] * (2 * NSLOT),
        compiler_params=pltpu.CompilerParams(use_tc_tiling_on_sc=False),
    )
    def k(u_hbm, srcb_hbm, dst_hbm, zrows_hbm, out_hbm,
          src_all, dst_all, *rest):
        rows = rest[:NSLOT]
        acc_sh = rest[NSLOT]
        gsem = rest[NSLOT + 1:2 * NSLOT + 1]
        ssem = rest[2 * NSLOT + 1:]
        c = lax.axis_index("c")
        s = lax.axis_index("s")
        row0 = s * stripe
        pltpu.sync_copy(zrows_hbm, acc_sh.at[pl.ds(row0, stripe)])
        cbase = s * cpt
        pltpu.sync_copy(srcb_hbm.at[c, pl.ds(cbase, cpt)], src_all)
        pltpu.sync_copy(dst_hbm.at[pl.ds(cbase, cpt)], dst_all)
        plsc.subcore_barrier()

        def fire_gather(i, slot):
            pltpu.async_copy(u_hbm.at[src_all.at[i]], rows[slot], gsem[slot])

        def wait_scatter(i, slot):
            pltpu.make_async_copy(
                rows[slot], acc_sh.at[dst_all.at[i]], ssem[slot]).wait()

        fire_gather(0, 0)
        fire_gather(1, 1)

        def body(kk, carry):
            for j in range(NSLOT):
                i = NSLOT * kk + j
                pj = (j + NSLOT - 1) % NSLOT
                pltpu.make_async_copy(
                    u_hbm.at[src_all.at[i]], rows[j], gsem[j]).wait()
                pltpu.async_copy(
                    rows[j], acc_sh.at[dst_all.at[i]], ssem[j], add=True)
                pl.when(i >= 1)(functools.partial(wait_scatter, i - 1, pj))
                pl.when(i + 2 < cpt)(functools.partial(fire_gather, i + 2, pj))
            return carry

        lax.fori_loop(0, cpt // NSLOT, body, 0)
        wait_scatter(cpt - 1, (cpt - 1) % NSLOT)
        plsc.subcore_barrier()
        pltpu.sync_copy(acc_sh.at[pl.ds(row0, stripe)],
                        out_hbm.at[c, pl.ds(row0, stripe)])

    return k


@functools.lru_cache(maxsize=None)
def _deg_sc(n_acc, cpt):
    stripe = n_acc // NS

    @functools.partial(
        pl.kernel,
        out_type=jax.ShapeDtypeStruct((NC, n_acc, DW), jnp.float32),
        mesh=_mesh(),
        scratch_types=[
            pltpu.VMEM((cpt, CHUNK), jnp.int32),
            pltpu.VMEM((CHUNK, DW), jnp.float32),
            pltpu.VMEM_SHARED((n_acc, DW), jnp.float32),
        ],
        compiler_params=pltpu.CompilerParams(use_tc_tiling_on_sc=False),
    )
    def k(dst_hbm, ones_hbm, zrows_hbm, out_hbm,
          dst_all, ones_v, acc_sh):
        c = lax.axis_index("c")
        s = lax.axis_index("s")
        wid = c * NS + s
        row0 = s * stripe
        pltpu.sync_copy(ones_hbm, ones_v)
        pltpu.sync_copy(zrows_hbm, acc_sh.at[pl.ds(row0, stripe)])
        pltpu.sync_copy(dst_hbm.at[pl.ds(wid * cpt, cpt)], dst_all)
        plsc.subcore_barrier()

        def body(i, carry):
            pltpu.sync_copy(ones_v, acc_sh.at[dst_all.at[i]], add=True)
            return carry

        lax.fori_loop(0, cpt, body, 0)
        plsc.subcore_barrier()
        pltpu.sync_copy(acc_sh.at[pl.ds(row0, stripe)],
                        out_hbm.at[c, pl.ds(row0, stripe)])

    return k


def _scale_kernel(d0, d1, x, dinv_o, u1_o):
    rb, d = x.shape
    dh = d // 2
    deg = d0[0, :, 0:1] + d1[0, :, 0:1] + 1.0
    dinv = jnp.broadcast_to(lax.rsqrt(deg), (rb, dh))
    xb = x[...]
    dinv_o[...] = jnp.stack([dinv, dinv])
    u1_o[...] = jnp.stack([dinv * xb[:, :dh], dinv * xb[:, dh:]])


def _hidden_kernel(p, u1, dinv, w1, b1, u2_o):
    dh = u1.shape[-1]
    a0 = dinv[0] * (p[0] + u1[0])
    a1 = dinv[1] * (p[1] + u1[1])
    h = (jnp.dot(a0, w1[:dh, :], preferred_element_type=jnp.float32)
         + jnp.dot(a1, w1[dh:, :], preferred_element_type=jnp.float32)
         + b1[...])
    h = jnp.maximum(h, 0.0)
    u2_o[...] = jnp.stack([dinv[0] * h[:, :dh], dinv[1] * h[:, dh:]])


def _head_kernel(q, u2, dinv, wmu, bmu, wlv, blv, mu_o, lv_o):
    dh = u2.shape[-1]
    a0 = dinv[0] * (q[0] + u2[0])
    a1 = dinv[1] * (q[1] + u2[1])
    mu_o[...] = (jnp.dot(a0, wmu[:dh, :], preferred_element_type=jnp.float32)
                 + jnp.dot(a1, wmu[dh:, :], preferred_element_type=jnp.float32)
                 + bmu[...])
    lv_o[...] = (jnp.dot(a0, wlv[:dh, :], preferred_element_type=jnp.float32)
                 + jnp.dot(a1, wlv[dh:, :], preferred_element_type=jnp.float32)
                 + blv[...])


def kernel(x, edge_index, W1, b1, Wmu, bmu, Wlv, blv):
    n, d_in = x.shape
    e = edge_index.shape[1]
    d_hid = W1.shape[1]
    d_out = Wmu.shape[1]
    dh = d_in // 2

    n_acc = ((n + 1 + NS * 8 - 1) // (NS * 8)) * (NS * 8)
    stripe = n_acc // NS
    del stripe
    cpt_deg = 3 * -(-e // (3 * NW * CHUNK))
    e_pad = cpt_deg * NW * CHUNK
    cpt_prop = e_pad // (NS * CHUNK)

    src = edge_index[0]
    dst = edge_index[1]
    pad = e_pad - e
    if pad:
        src = jnp.concatenate([src, jnp.zeros((pad,), jnp.int32)])
        dst = jnp.concatenate([dst, jnp.full((pad,), n, jnp.int32)])
    src_both = jnp.stack([src, src + n]).reshape(NC, e_pad // CHUNK, CHUNK)
    dst = dst.reshape(e_pad // CHUNK, CHUNK)

    ones_w = jnp.ones((CHUNK, DW), jnp.float32)
    zrows_w = jnp.zeros((n_acc // NS, DW), jnp.float32)
    zrows_d = jnp.zeros((n_acc // NS, dh), jnp.float32)

    degp = _deg_sc(n_acc, cpt_deg)(dst, ones_w, zrows_w)

    rb = 1000 if n % 1000 == 0 else 8
    grid = (n // rb,)
    rowspec = lambda width: pl.BlockSpec((rb, width), lambda i: (i, 0))
    stspec = lambda width: pl.BlockSpec((NC, rb, width), lambda i: (0, i, 0))
    fullspec = lambda a, b: pl.BlockSpec((a, b), lambda i: (0, 0))

    dinv_st, u1_st = pl.pallas_call(
        _scale_kernel,
        grid=grid,
        in_specs=[pl.BlockSpec((1, rb, DW), lambda i: (0, i, 0)),
                  pl.BlockSpec((1, rb, DW), lambda i: (1, i, 0)),
                  rowspec(d_in)],
        out_specs=[stspec(dh), stspec(dh)],
        out_shape=[jax.ShapeDtypeStruct((NC, n, dh), jnp.float32)] * 2,
    )(degp, degp, x)

    prop = _prop_sc(n_acc, dh, cpt_prop)
    p = prop(u1_st.reshape(NC * n, dh), src_both, dst, zrows_d)

    u2_st = pl.pallas_call(
        _hidden_kernel,
        grid=grid,
        in_specs=[stspec(dh)] * 3 + [fullspec(d_in, d_hid), fullspec(1, d_hid)],
        out_specs=stspec(dh),
        out_shape=jax.ShapeDtypeStruct((NC, n, dh), jnp.float32),
    )(p, u1_st, dinv_st, W1, b1.reshape(1, d_hid))

    q = prop(u2_st.reshape(NC * n, dh), src_both, dst, zrows_d)

    mu, lv = pl.pallas_call(
        _head_kernel,
        grid=grid,
        in_specs=[stspec(dh)] * 3
        + [fullspec(d_hid, d_out), fullspec(1, d_out),
           fullspec(d_hid, d_out), fullspec(1, d_out)],
        out_specs=[rowspec(d_out), rowspec(d_out)],
        out_shape=[jax.ShapeDtypeStruct((n, d_out), jnp.float32)] * 2,
    )(q, u2_st, dinv_st, Wmu, bmu.reshape(1, d_out),
      Wlv, blv.reshape(1, d_out))

    return mu, lv


# speedup vs baseline: 1.8196x; 1.8196x over previous
import functools

import jax
import jax.numpy as jnp
from jax import lax
from jax.experimental import pallas as pl
from jax.experimental.pallas import tpu as pltpu
from jax.experimental.pallas import tpu_sc as plsc

NC = 2
NS = 16
NW = NC * NS
CHUNK = 128
DW = 16


def _mesh():
    return plsc.VectorSubcoreMesh(core_axis_name="c", subcore_axis_name="s")


@functools.lru_cache(maxsize=None)
def _prop_sc(n_acc, dh, cpt):
    stripe = n_acc // NS
    NSLOT = 2
    assert cpt % NSLOT == 0 and cpt > NSLOT

    @functools.partial(
        pl.kernel,
        out_type=jax.ShapeDtypeStruct((NC, n_acc, dh), jnp.float32),
        mesh=_mesh(),
        scratch_types=[
            pltpu.VMEM((cpt, CHUNK), jnp.int32),
            pltpu.VMEM((cpt, CHUNK), jnp.int32),
        ]
        + [pltpu.VMEM((CHUNK, dh), jnp.float32)] * NSLOT
        + [pltpu.VMEM_SHARED((n_acc, dh), jnp.float32)]
        + [pltpu.SemaphoreType.DMA] * NSLOT,
        compiler_params=pltpu.CompilerParams(use_tc_tiling_on_sc=False),
    )
    def k(u_hbm, srcb_hbm, dst_hbm, zrows_hbm, out_hbm,
          src_all, dst_all, *rest):
        rows = rest[:NSLOT]
        acc_sh = rest[NSLOT]
        sems = rest[NSLOT + 1:]
        c = lax.axis_index("c")
        s = lax.axis_index("s")
        row0 = s * stripe
        pltpu.sync_copy(zrows_hbm, acc_sh.at[pl.ds(row0, stripe)])
        cbase = s * cpt
        pltpu.sync_copy(srcb_hbm.at[c, pl.ds(cbase, cpt)], src_all)
        pltpu.sync_copy(dst_hbm.at[pl.ds(cbase, cpt)], dst_all)
        plsc.subcore_barrier()

        def fire(i, slot):
            pltpu.async_copy(u_hbm.at[src_all.at[i]], rows[slot], sems[slot])

        for j in range(NSLOT - 1):
            fire(j, j)

        def body(kk, carry):
            for j in range(NSLOT):
                i = NSLOT * kk + j
                nslot = (j + NSLOT - 1) % NSLOT
                pl.when(i + NSLOT - 1 < cpt)(
                    functools.partial(fire, i + NSLOT - 1, nslot))
                pltpu.make_async_copy(
                    u_hbm.at[src_all.at[i]], rows[j], sems[j]).wait()
                pltpu.sync_copy(rows[j], acc_sh.at[dst_all.at[i]], add=True)
            return carry

        lax.fori_loop(0, cpt // NSLOT, body, 0)
        plsc.subcore_barrier()
        pltpu.sync_copy(acc_sh.at[pl.ds(row0, stripe)],
                        out_hbm.at[c, pl.ds(row0, stripe)])

    return k


@functools.lru_cache(maxsize=None)
def _deg_sc(n_acc, cpt):
    stripe = n_acc // NS

    @functools.partial(
        pl.kernel,
        out_type=jax.ShapeDtypeStruct((NC, n_acc, DW), jnp.float32),
        mesh=_mesh(),
        scratch_types=[
            pltpu.VMEM((cpt, CHUNK), jnp.int32),
            pltpu.VMEM((CHUNK, DW), jnp.float32),
            pltpu.VMEM_SHARED((n_acc, DW), jnp.float32),
        ],
        compiler_params=pltpu.CompilerParams(use_tc_tiling_on_sc=False),
    )
    def k(dst_hbm, ones_hbm, zrows_hbm, out_hbm,
          dst_all, ones_v, acc_sh):
        c = lax.axis_index("c")
        s = lax.axis_index("s")
        wid = c * NS + s
        row0 = s * stripe
        pltpu.sync_copy(ones_hbm, ones_v)
        pltpu.sync_copy(zrows_hbm, acc_sh.at[pl.ds(row0, stripe)])
        pltpu.sync_copy(dst_hbm.at[pl.ds(wid * cpt, cpt)], dst_all)
        plsc.subcore_barrier()

        def body(i, carry):
            pltpu.sync_copy(ones_v, acc_sh.at[dst_all.at[i]], add=True)
            return carry

        lax.fori_loop(0, cpt, body, 0)
        plsc.subcore_barrier()
        pltpu.sync_copy(acc_sh.at[pl.ds(row0, stripe)],
                        out_hbm.at[c, pl.ds(row0, stripe)])

    return k


def _scale_kernel(d0, d1, x, dinv_o, u1_o):
    rb, d = x.shape
    dh = d // 2
    deg = d0[0, :, 0:1] + d1[0, :, 0:1] + 1.0
    dinv = jnp.broadcast_to(lax.rsqrt(deg), (rb, dh))
    xb = x[...]
    dinv_o[...] = jnp.stack([dinv, dinv])
    u1_o[...] = jnp.stack([dinv * xb[:, :dh], dinv * xb[:, dh:]])


def _hidden_kernel(p, u1, dinv, w1, b1, u2_o):
    dh = u1.shape[-1]
    a0 = dinv[0] * (p[0] + u1[0])
    a1 = dinv[1] * (p[1] + u1[1])
    h = (jnp.dot(a0, w1[:dh, :], preferred_element_type=jnp.float32)
         + jnp.dot(a1, w1[dh:, :], preferred_element_type=jnp.float32)
         + b1[...])
    h = jnp.maximum(h, 0.0)
    u2_o[...] = jnp.stack([dinv[0] * h[:, :dh], dinv[1] * h[:, dh:]])


def _head_kernel(q, u2, dinv, wmu, bmu, wlv, blv, mu_o, lv_o):
    dh = u2.shape[-1]
    a0 = dinv[0] * (q[0] + u2[0])
    a1 = dinv[1] * (q[1] + u2[1])
    mu_o[...] = (jnp.dot(a0, wmu[:dh, :], preferred_element_type=jnp.float32)
                 + jnp.dot(a1, wmu[dh:, :], preferred_element_type=jnp.float32)
                 + bmu[...])
    lv_o[...] = (jnp.dot(a0, wlv[:dh, :], preferred_element_type=jnp.float32)
                 + jnp.dot(a1, wlv[dh:, :], preferred_element_type=jnp.float32)
                 + blv[...])


def kernel(x, edge_index, W1, b1, Wmu, bmu, Wlv, blv):
    n, d_in = x.shape
    e = edge_index.shape[1]
    d_hid = W1.shape[1]
    d_out = Wmu.shape[1]
    dh = d_in // 2

    n_acc = ((n + 1 + NS * 8 - 1) // (NS * 8)) * (NS * 8)
    stripe = n_acc // NS
    del stripe
    cpt_deg = -(-e // (NW * CHUNK))
    e_pad = cpt_deg * NW * CHUNK
    cpt_prop = e_pad // (NS * CHUNK)

    src = edge_index[0]
    dst = edge_index[1]
    pad = e_pad - e
    if pad:
        src = jnp.concatenate([src, jnp.zeros((pad,), jnp.int32)])
        dst = jnp.concatenate([dst, jnp.full((pad,), n, jnp.int32)])
    src_both = jnp.stack([src, src + n]).reshape(NC, e_pad // CHUNK, CHUNK)
    dst = dst.reshape(e_pad // CHUNK, CHUNK)

    ones_w = jnp.ones((CHUNK, DW), jnp.float32)
    zrows_w = jnp.zeros((n_acc // NS, DW), jnp.float32)
    zrows_d = jnp.zeros((n_acc // NS, dh), jnp.float32)

    degp = _deg_sc(n_acc, cpt_deg)(dst, ones_w, zrows_w)

    rb = 1000 if n % 1000 == 0 else 8
    grid = (n // rb,)
    rowspec = lambda width: pl.BlockSpec((rb, width), lambda i: (i, 0))
    stspec = lambda width: pl.BlockSpec((NC, rb, width), lambda i: (0, i, 0))
    fullspec = lambda a, b: pl.BlockSpec((a, b), lambda i: (0, 0))

    dinv_st, u1_st = pl.pallas_call(
        _scale_kernel,
        grid=grid,
        in_specs=[pl.BlockSpec((1, rb, DW), lambda i: (0, i, 0)),
                  pl.BlockSpec((1, rb, DW), lambda i: (1, i, 0)),
                  rowspec(d_in)],
        out_specs=[stspec(dh), stspec(dh)],
        out_shape=[jax.ShapeDtypeStruct((NC, n, dh), jnp.float32)] * 2,
    )(degp, degp, x)

    prop = _prop_sc(n_acc, dh, cpt_prop)
    p = prop(u1_st.reshape(NC * n, dh), src_both, dst, zrows_d)

    u2_st = pl.pallas_call(
        _hidden_kernel,
        grid=grid,
        in_specs=[stspec(dh)] * 3 + [fullspec(d_in, d_hid), fullspec(1, d_hid)],
        out_specs=stspec(dh),
        out_shape=jax.ShapeDtypeStruct((NC, n, dh), jnp.float32),
    )(p, u1_st, dinv_st, W1, b1.reshape(1, d_hid))

    q = prop(u2_st.reshape(NC * n, dh), src_both, dst, zrows_d)

    mu, lv = pl.pallas_call(
        _head_kernel,
        grid=grid,
        in_specs=[stspec(dh)] * 3
        + [fullspec(d_hid, d_out), fullspec(1, d_out),
           fullspec(d_hid, d_out), fullspec(1, d_out)],
        out_specs=[rowspec(d_out), rowspec(d_out)],
        out_shape=[jax.ShapeDtypeStruct((n, d_out), jnp.float32)] * 2,
    )(q, u2_st, dinv_st, Wmu, bmu.reshape(1, d_out),
      Wlv, blv.reshape(1, d_out))

    return mu, lv


# speedup vs baseline: 1.8197x; 1.0001x over previous
import functools

import jax
import jax.numpy as jnp
from jax import lax
from jax.experimental import pallas as pl
from jax.experimental.pallas import tpu as pltpu
from jax.experimental.pallas import tpu_sc as plsc

NC = 2
NS = 16
NW = NC * NS
CHUNK = 128
DW = 16


def _mesh():
    return plsc.VectorSubcoreMesh(core_axis_name="c", subcore_axis_name="s")


@functools.lru_cache(maxsize=None)
def _prop_sc(n_acc, dh, cpt):
    stripe = n_acc // NS
    NSLOT = 2
    assert cpt % NSLOT == 0 and cpt > NSLOT

    @functools.partial(
        pl.kernel,
        out_type=jax.ShapeDtypeStruct((NC, n_acc, dh), jnp.float32),
        mesh=_mesh(),
        scratch_types=[
            pltpu.VMEM((cpt, CHUNK), jnp.int32),
            pltpu.VMEM((cpt, CHUNK), jnp.int32),
        ]
        + [pltpu.VMEM((CHUNK, dh), jnp.float32)] * NSLOT
        + [pltpu.VMEM_SHARED((n_acc, dh), jnp.float32)]
        + [pltpu.SemaphoreType.DMA] * NSLOT,
        compiler_params=pltpu.CompilerParams(use_tc_tiling_on_sc=False),
    )
    def k(u_hbm, srcb_hbm, dst_hbm, zrows_hbm, out_hbm,
          src_all, dst_all, *rest):
        rows = rest[:NSLOT]
        acc_sh = rest[NSLOT]
        sems = rest[NSLOT + 1:]
        c = lax.axis_index("c")
        s = lax.axis_index("s")
        row0 = s * stripe
        pltpu.sync_copy(zrows_hbm, acc_sh.at[pl.ds(row0, stripe)])
        cbase = s * cpt
        pltpu.sync_copy(srcb_hbm.at[c, pl.ds(cbase, cpt)], src_all)
        pltpu.sync_copy(dst_hbm.at[pl.ds(cbase, cpt)], dst_all)
        plsc.subcore_barrier()

        def fire(i, slot):
            pltpu.async_copy(u_hbm.at[src_all.at[i]], rows[slot], sems[slot])

        for j in range(NSLOT - 1):
            fire(j, j)

        def body(kk, carry):
            for j in range(NSLOT):
                i = NSLOT * kk + j
                nslot = (j + NSLOT - 1) % NSLOT
                pl.when(i + NSLOT - 1 < cpt)(
                    functools.partial(fire, i + NSLOT - 1, nslot))
                pltpu.make_async_copy(
                    u_hbm.at[src_all.at[i]], rows[j], sems[j]).wait()
                pltpu.sync_copy(rows[j], acc_sh.at[dst_all.at[i]], add=True)
            return carry

        lax.fori_loop(0, cpt // NSLOT, body, 0)
        plsc.subcore_barrier()
        pltpu.sync_copy(acc_sh.at[pl.ds(row0, stripe)],
                        out_hbm.at[c, pl.ds(row0, stripe)])

    return k


@functools.lru_cache(maxsize=None)
def _deg_sc(n_acc, cpt):
    stripe = n_acc // NS

    @functools.partial(
        pl.kernel,
        out_type=jax.ShapeDtypeStruct((NC, n_acc, DW), jnp.float32),
        mesh=_mesh(),
        scratch_types=[
            pltpu.VMEM((cpt, CHUNK), jnp.int32),
            pltpu.VMEM((CHUNK, DW), jnp.float32),
            pltpu.VMEM_SHARED((n_acc, DW), jnp.float32),
        ],
        compiler_params=pltpu.CompilerParams(use_tc_tiling_on_sc=False),
    )
    def k(dst_hbm, ones_hbm, zrows_hbm, out_hbm,
          dst_all, ones_v, acc_sh):
        c = lax.axis_index("c")
        s = lax.axis_index("s")
        wid = c * NS + s
        row0 = s * stripe
        pltpu.sync_copy(ones_hbm, ones_v)
        pltpu.sync_copy(zrows_hbm, acc_sh.at[pl.ds(row0, stripe)])
        pltpu.sync_copy(dst_hbm.at[pl.ds(wid * cpt, cpt)], dst_all)
        plsc.subcore_barrier()

        def body(i, carry):
            pltpu.sync_copy(ones_v, acc_sh.at[dst_all.at[i]], add=True)
            return carry

        lax.fori_loop(0, cpt, body, 0)
        plsc.subcore_barrier()
        pltpu.sync_copy(acc_sh.at[pl.ds(row0, stripe)],
                        out_hbm.at[c, pl.ds(row0, stripe)])

    return k


def _scale_kernel(d0, d1, x, dinv_o, u1_o):
    rb, d = x.shape
    dh = d // 2
    deg = d0[0, :, 0:1] + d1[0, :, 0:1] + 1.0
    dinv = jnp.broadcast_to(lax.rsqrt(deg), (rb, dh))
    xb = x[...]
    dinv_o[...] = jnp.stack([dinv, dinv])
    u1_o[...] = jnp.stack([dinv * xb[:, :dh], dinv * xb[:, dh:]])


def _hidden_kernel(p, u1, dinv, w1, b1, u2_o):
    dh = u1.shape[-1]
    a0 = dinv[0] * (p[0] + u1[0])
    a1 = dinv[1] * (p[1] + u1[1])
    h = (jnp.dot(a0, w1[:dh, :], preferred_element_type=jnp.float32)
         + jnp.dot(a1, w1[dh:, :], preferred_element_type=jnp.float32)
         + b1[...])
    h = jnp.maximum(h, 0.0)
    u2_o[...] = jnp.stack([dinv[0] * h[:, :dh], dinv[1] * h[:, dh:]])


def _head_kernel(q, u2, dinv, wmu, bmu, wlv, blv, mu_o, lv_o):
    dh = u2.shape[-1]
    a0 = dinv[0] * (q[0] + u2[0])
    a1 = dinv[1] * (q[1] + u2[1])
    mu_o[...] = (jnp.dot(a0, wmu[:dh, :], preferred_element_type=jnp.float32)
                 + jnp.dot(a1, wmu[dh:, :], preferred_element_type=jnp.float32)
                 + bmu[...])
    lv_o[...] = (jnp.dot(a0, wlv[:dh, :], preferred_element_type=jnp.float32)
                 + jnp.dot(a1, wlv[dh:, :], preferred_element_type=jnp.float32)
                 + blv[...])


def kernel(x, edge_index, W1, b1, Wmu, bmu, Wlv, blv):
    n, d_in = x.shape
    e = edge_index.shape[1]
    d_hid = W1.shape[1]
    d_out = Wmu.shape[1]
    dh = d_in // 2

    n_acc = ((n + 1 + NS * 8 - 1) // (NS * 8)) * (NS * 8)
    cpt_deg = -(-e // (NW * CHUNK))
    e_pad = cpt_deg * NW * CHUNK
    cpt_prop = e_pad // (NS * CHUNK)

    src = edge_index[0]
    dst = edge_index[1]
    pad = e_pad - e
    if pad:
        src = jnp.concatenate([src, jnp.zeros((pad,), jnp.int32)])
        dst = jnp.concatenate([dst, jnp.full((pad,), n, jnp.int32)])
    src_both = jnp.stack([src, src + n]).reshape(NC, e_pad // CHUNK, CHUNK)
    dst = dst.reshape(e_pad // CHUNK, CHUNK)

    ones_w = jnp.ones((CHUNK, DW), jnp.float32)
    zrows_w = jnp.zeros((n_acc // NS, DW), jnp.float32)
    zrows_d = jnp.zeros((n_acc // NS, dh), jnp.float32)

    degp = _deg_sc(n_acc, cpt_deg)(dst, ones_w, zrows_w)

    rb = 1000 if n % 1000 == 0 else 8
    grid = (n // rb,)
    rowspec = lambda width: pl.BlockSpec((rb, width), lambda i: (i, 0))
    stspec = lambda width: pl.BlockSpec((NC, rb, width), lambda i: (0, i, 0))
    fullspec = lambda a, b: pl.BlockSpec((a, b), lambda i: (0, 0))

    dinv_st, u1_st = pl.pallas_call(
        _scale_kernel,
        grid=grid,
        in_specs=[pl.BlockSpec((1, rb, DW), lambda i: (0, i, 0)),
                  pl.BlockSpec((1, rb, DW), lambda i: (1, i, 0)),
                  rowspec(d_in)],
        out_specs=[stspec(dh), stspec(dh)],
        out_shape=[jax.ShapeDtypeStruct((NC, n, dh), jnp.float32)] * 2,
    )(degp, degp, x)

    prop = _prop_sc(n_acc, dh, cpt_prop)
    p = prop(u1_st.reshape(NC * n, dh), src_both, dst, zrows_d)

    u2_st = pl.pallas_call(
        _hidden_kernel,
        grid=grid,
        in_specs=[stspec(dh)] * 3 + [fullspec(d_in, d_hid), fullspec(1, d_hid)],
        out_specs=stspec(dh),
        out_shape=jax.ShapeDtypeStruct((NC, n, dh), jnp.float32),
    )(p, u1_st, dinv_st, W1, b1.reshape(1, d_hid))

    q = prop(u2_st.reshape(NC * n, dh), src_both, dst, zrows_d)

    mu, lv = pl.pallas_call(
        _head_kernel,
        grid=grid,
        in_specs=[stspec(dh)] * 3
        + [fullspec(d_hid, d_out), fullspec(1, d_out),
           fullspec(d_hid, d_out), fullspec(1, d_out)],
        out_specs=[rowspec(d_out), rowspec(d_out)],
        out_shape=[jax.ShapeDtypeStruct((n, d_out), jnp.float32)] * 2,
    )(q, u2_st, dinv_st, Wmu, bmu.reshape(1, d_out),
      Wlv, blv.reshape(1, d_out))

    return mu, lv
